# Initial kernel scaffold; baseline (speedup 1.0000x reference)
import jax
import jax.numpy as jnp
from jax.experimental import pallas as pl


def kernel(pred_loc, pred_cls, label):
    raise NotImplementedError("write your pallas kernel here")


# speedup vs baseline: 14.5362x; 14.5362x over previous
import functools
from math import sqrt

import jax
import jax.numpy as jnp
import numpy as np
from jax import lax
from jax.experimental import pallas as pl
from jax.experimental.pallas import tpu as pltpu

THRESHOLD = 0.5
NEG_POS_RATIO = 3
ALPHA = 1.0

B = 32
P = 8732
C = 21
M = 12
SUB = 8
LN = 1152
PP = SUB * LN

_INTERPRET = False


def _priors_cxcy_np():
    fmap_dims = {'conv4_3': 38, 'conv7': 19, 'conv8_2': 10, 'conv9_2': 5, 'conv10_2': 3, 'conv11_2': 1}
    obj_scales = {'conv4_3': 0.1, 'conv7': 0.2, 'conv8_2': 0.375, 'conv9_2': 0.55, 'conv10_2': 0.725, 'conv11_2': 0.9}
    aspect_ratios = {'conv4_3': [1.0, 2.0, 0.5], 'conv7': [1.0, 2.0, 3.0, 0.5, 0.333], 'conv8_2': [1.0, 2.0, 3.0, 0.5, 0.333], 'conv9_2': [1.0, 2.0, 3.0, 0.5, 0.333], 'conv10_2': [1.0, 2.0, 0.5], 'conv11_2': [1.0, 2.0, 0.5]}
    fmaps = list(fmap_dims.keys())
    pb = []
    for k, fmap in enumerate(fmaps):
        for i in range(fmap_dims[fmap]):
            for j in range(fmap_dims[fmap]):
                cx = (j + 0.5) / fmap_dims[fmap]
                cy = (i + 0.5) / fmap_dims[fmap]
                for ratio in aspect_ratios[fmap]:
                    pb.append([cx, cy, obj_scales[fmap] * sqrt(ratio), obj_scales[fmap] / sqrt(ratio)])
                    if ratio == 1.0:
                        try:
                            additional_scale = sqrt(obj_scales[fmap] * obj_scales[fmaps[k + 1]])
                        except IndexError:
                            additional_scale = 1.0
                        pb.append([cx, cy, additional_scale, additional_scale])
    return np.clip(np.array(pb, dtype=np.float32), 0.0, 1.0)


def _prior_planes():
    cxcy = _priors_cxcy_np()
    cx, cy, w, h = cxcy[:, 0], cxcy[:, 1], cxcy[:, 2], cxcy[:, 3]
    x1 = cx - w / 2.0
    y1 = cy - h / 2.0
    x2 = cx + w / 2.0
    y2 = cy + h / 2.0
    area = (x2 - x1) * (y2 - y1)
    planes = np.stack([x1, y1, x2, y2, area, cx, cy,
                       10.0 / w, 10.0 / h, 5.0 * np.log(w), 5.0 * np.log(h)], axis=0)
    pad_vals = np.array([2.0, 2.0, 2.0, 2.0, 0.0, 2.0, 2.0, 10.0, 10.0, 0.0, 0.0],
                        dtype=np.float32)
    padded = np.tile(pad_vals[:, None], (1, PP)).astype(np.float32)
    padded[:, :P] = planes
    return jnp.asarray(padded.reshape(11, SUB, LN).reshape(11 * SUB, LN))


_PR_PLANES = _prior_planes()


def _body(pr_ref, lbl_ref, ploc_ref, pcls_ref, out_ref, acc_ref):
    b = pl.program_id(0)

    @pl.when(b == 0)
    def _init():
        acc_ref[0] = 0.0
        acc_ref[1] = 0.0
        acc_ref[2] = 0.0
        acc_ref[3] = 0.0

    def plane(i):
        return pr_ref[8 * i:8 * i + 8, :]

    PX1, PY1, PX2, PY2 = plane(0), plane(1), plane(2), plane(3)
    PAREA = plane(4)
    PCX, PCY = plane(5), plane(6)
    PIW10, PIH10 = plane(7), plane(8)
    PLW5, PLH5 = plane(9), plane(10)

    pidx = (lax.broadcasted_iota(jnp.int32, (SUB, LN), 0) * LN
            + lax.broadcasted_iota(jnp.int32, (SUB, LN), 1))

    def lbl(m, c):
        return lbl_ref[0, 0, 5 * m + c]

    obj = jnp.zeros((SUB, LN), jnp.int32)
    curmax = jnp.full((SUB, LN), -1.0, jnp.float32)
    row_arg = []
    for m in range(M):
        bx1, by1, bx2, by2 = lbl(m, 0), lbl(m, 1), lbl(m, 2), lbl(m, 3)
        iw = jnp.maximum(jnp.minimum(bx2, PX2) - jnp.maximum(bx1, PX1), 0.0)
        ih = jnp.maximum(jnp.minimum(by2, PY2) - jnp.maximum(by1, PY1), 0.0)
        inter = iw * ih
        barea = (bx2 - bx1) * (by2 - by1)
        iou = inter / (barea + PAREA - inter)
        upd = iou > curmax
        obj = jnp.where(upd, m, obj)
        curmax = jnp.where(upd, iou, curmax)
        rm = jnp.max(iou)
        cand = jnp.where(iou == rm, pidx, jnp.int32(PP))
        row_arg.append(jnp.min(cand))

    for m in range(M):
        msk = pidx == row_arg[m]
        obj = jnp.where(msk, m, obj)
        curmax = jnp.where(msk, 1.0, curmax)

    labp = jnp.zeros((SUB, LN), jnp.float32)
    gx1 = jnp.zeros((SUB, LN), jnp.float32)
    gy1 = jnp.zeros((SUB, LN), jnp.float32)
    gx2 = jnp.zeros((SUB, LN), jnp.float32)
    gy2 = jnp.zeros((SUB, LN), jnp.float32)
    for m in range(M):
        sel = obj == m
        labp = jnp.where(sel, lbl(m, 4), labp)
        gx1 = jnp.where(sel, lbl(m, 0), gx1)
        gy1 = jnp.where(sel, lbl(m, 1), gy1)
        gx2 = jnp.where(sel, lbl(m, 2), gx2)
        gy2 = jnp.where(sel, lbl(m, 3), gy2)
    labp = jnp.where(curmax < THRESHOLD, 0.0, labp)

    pos = labp != 0.0
    posf = pos.astype(jnp.float32)
    n_pos_i = jnp.sum(pos.astype(jnp.int32))

    bcx = (gx1 + gx2) * 0.5
    bcy = (gy1 + gy2) * 0.5
    bw = gx2 - gx1
    bh = gy2 - gy1
    t0 = (bcx - PCX) * PIW10
    t1 = (bcy - PCY) * PIH10
    t2 = jnp.log(bw) * 5.0 - PLW5
    t3 = jnp.log(bh) * 5.0 - PLH5
    loc_l1 = (jnp.sum(jnp.abs(ploc_ref[0, 0:8, :] - t0) * posf)
              + jnp.sum(jnp.abs(ploc_ref[0, 8:16, :] - t1) * posf)
              + jnp.sum(jnp.abs(ploc_ref[0, 16:24, :] - t2) * posf)
              + jnp.sum(jnp.abs(ploc_ref[0, 24:32, :] - t3) * posf))

    def cls_plane(c):
        return pcls_ref[0, 8 * c:8 * c + 8, :]

    mx = cls_plane(0)
    for c in range(1, C):
        mx = jnp.maximum(mx, cls_plane(c))
    s = jnp.exp(cls_plane(0) - mx)
    for c in range(1, C):
        s = s + jnp.exp(cls_plane(c) - mx)
    logz = mx + jnp.log(s)
    pick = cls_plane(0)
    for c in range(1, C):
        pick = jnp.where(labp == jnp.float32(c), cls_plane(c), pick)
    conf = logz - pick
    conf_pos_sum = jnp.sum(conf * posf)

    valid = pidx < P
    neg = jnp.where(pos | (~valid), 0.0, conf)

    bits = lax.bitcast_convert_type(neg, jnp.int32)
    ki = NEG_POS_RATIO * n_pos_i

    def bit_step(i, t):
        cand = t | lax.shift_left(jnp.int32(1), 30 - i)
        cnt = jnp.sum((bits >= cand).astype(jnp.int32))
        return jnp.where(cnt >= ki, cand, t)

    tbits = lax.fori_loop(0, 31, bit_step, jnp.int32(0))
    tval = lax.bitcast_convert_type(tbits, jnp.float32)
    gt = bits > tbits
    cgt = jnp.sum(gt.astype(jnp.int32))
    sgt = jnp.sum(jnp.where(gt, neg, 0.0))
    hard_sum = sgt + (ki - cgt).astype(jnp.float32) * tval

    acc_ref[0] = acc_ref[0] + loc_l1
    acc_ref[1] = acc_ref[1] + conf_pos_sum
    acc_ref[2] = acc_ref[2] + hard_sum
    acc_ref[3] = acc_ref[3] + n_pos_i.astype(jnp.float32)

    @pl.when(b == B - 1)
    def _fin():
        npos_t = acc_ref[3]
        out_ref[0, 0] = ((acc_ref[2] + acc_ref[1]) / npos_t
                         + ALPHA * acc_ref[0] / (4.0 * npos_t))


@jax.jit
def kernel(pred_loc, pred_cls, label):
    pl_t = jnp.transpose(pred_loc, (0, 2, 1))
    pl_r = jnp.pad(pl_t, ((0, 0), (0, 0), (0, PP - P)))
    pl_r = pl_r.reshape(B, 4 * SUB, LN)
    pc_t = jnp.transpose(pred_cls, (0, 2, 1))
    pc_r = jnp.pad(pc_t, ((0, 0), (0, 0), (0, PP - P)))
    pc_r = pc_r.reshape(B, C * SUB, LN)
    lbl_flat = label.reshape(B, 1, M * 5)

    out = pl.pallas_call(
        _body,
        grid=(B,),
        in_specs=[
            pl.BlockSpec((11 * SUB, LN), lambda b: (0, 0)),
            pl.BlockSpec((1, 1, M * 5), lambda b: (b, 0, 0), memory_space=pltpu.SMEM),
            pl.BlockSpec((1, 4 * SUB, LN), lambda b: (b, 0, 0)),
            pl.BlockSpec((1, C * SUB, LN), lambda b: (b, 0, 0)),
        ],
        out_specs=pl.BlockSpec((1, 1), lambda b: (0, 0), memory_space=pltpu.SMEM),
        out_shape=jax.ShapeDtypeStruct((1, 1), jnp.float32),
        scratch_shapes=[pltpu.SMEM((4,), jnp.float32)],
        compiler_params=pltpu.CompilerParams(
            dimension_semantics=("arbitrary",)),
        interpret=_INTERPRET,
    )(_PR_PLANES, lbl_flat, pl_r, pc_r)
    return out[0, 0]


# speedup vs baseline: 34.1779x; 2.3512x over previous
import functools
from math import sqrt

import jax
import jax.numpy as jnp
import numpy as np
from jax import lax
from jax.experimental import pallas as pl
from jax.experimental.pallas import tpu as pltpu

THRESHOLD = 0.5
NEG_POS_RATIO = 3
ALPHA = 1.0

B = 32
P = 8732
C = 21
M = 12
SUB = 8
LN = 1152
PP = SUB * LN

_INTERPRET = False


def _priors_cxcy_np():
    fmap_dims = {'conv4_3': 38, 'conv7': 19, 'conv8_2': 10, 'conv9_2': 5, 'conv10_2': 3, 'conv11_2': 1}
    obj_scales = {'conv4_3': 0.1, 'conv7': 0.2, 'conv8_2': 0.375, 'conv9_2': 0.55, 'conv10_2': 0.725, 'conv11_2': 0.9}
    aspect_ratios = {'conv4_3': [1.0, 2.0, 0.5], 'conv7': [1.0, 2.0, 3.0, 0.5, 0.333], 'conv8_2': [1.0, 2.0, 3.0, 0.5, 0.333], 'conv9_2': [1.0, 2.0, 3.0, 0.5, 0.333], 'conv10_2': [1.0, 2.0, 0.5], 'conv11_2': [1.0, 2.0, 0.5]}
    fmaps = list(fmap_dims.keys())
    pb = []
    for k, fmap in enumerate(fmaps):
        for i in range(fmap_dims[fmap]):
            for j in range(fmap_dims[fmap]):
                cx = (j + 0.5) / fmap_dims[fmap]
                cy = (i + 0.5) / fmap_dims[fmap]
                for ratio in aspect_ratios[fmap]:
                    pb.append([cx, cy, obj_scales[fmap] * sqrt(ratio), obj_scales[fmap] / sqrt(ratio)])
                    if ratio == 1.0:
                        try:
                            additional_scale = sqrt(obj_scales[fmap] * obj_scales[fmaps[k + 1]])
                        except IndexError:
                            additional_scale = 1.0
                        pb.append([cx, cy, additional_scale, additional_scale])
    return np.clip(np.array(pb, dtype=np.float32), 0.0, 1.0)


def _prior_planes():
    cxcy = _priors_cxcy_np()
    cx, cy, w, h = cxcy[:, 0], cxcy[:, 1], cxcy[:, 2], cxcy[:, 3]
    x1 = cx - w / 2.0
    y1 = cy - h / 2.0
    x2 = cx + w / 2.0
    y2 = cy + h / 2.0
    area = (x2 - x1) * (y2 - y1)
    planes = np.stack([x1, y1, x2, y2, area, cx, cy,
                       10.0 / w, 10.0 / h, 5.0 * np.log(w), 5.0 * np.log(h)], axis=0)
    pad_vals = np.array([2.0, 2.0, 2.0, 2.0, 0.0, 2.0, 2.0, 10.0, 10.0, 0.0, 0.0],
                        dtype=np.float32)
    padded = np.tile(pad_vals[:, None], (1, PP)).astype(np.float32)
    padded[:, :P] = planes
    return jnp.asarray(padded.reshape(11, SUB, LN).reshape(11 * SUB, LN))


_PR_PLANES = _prior_planes()


def _body(pr_ref, lbl_ref, ploc_ref, pcls_ref, out_ref,
          conf_s, labp_s, accloc_s):
    b = pl.program_id(0)

    def plane(i):
        return pr_ref[8 * i:8 * i + 8, :]

    PX1, PY1, PX2, PY2 = plane(0), plane(1), plane(2), plane(3)
    PAREA = plane(4)
    PCX, PCY = plane(5), plane(6)
    PIW10, PIH10 = plane(7), plane(8)
    PLW5, PLH5 = plane(9), plane(10)

    pidx = (lax.broadcasted_iota(jnp.int32, (SUB, LN), 0) * LN
            + lax.broadcasted_iota(jnp.int32, (SUB, LN), 1))

    def lbl(m, c):
        return lbl_ref[0, 0, 5 * m + c]

    iou_rows = []
    for m in range(M):
        bx1, by1, bx2, by2 = lbl(m, 0), lbl(m, 1), lbl(m, 2), lbl(m, 3)
        iw = jnp.maximum(jnp.minimum(bx2, PX2) - jnp.maximum(bx1, PX1), 0.0)
        ih = jnp.maximum(jnp.minimum(by2, PY2) - jnp.maximum(by1, PY1), 0.0)
        inter = iw * ih
        barea = (bx2 - bx1) * (by2 - by1)
        iou_rows.append(inter / (barea + PAREA - inter))
    iou3 = jnp.stack(iou_rows, axis=0)

    midx = lax.broadcasted_iota(jnp.int32, (M, SUB, LN), 0)
    pidx3 = pidx[None, :, :]

    curmax = jnp.max(iou3, axis=0)
    obj = jnp.min(jnp.where(iou3 == curmax[None], midx, M), axis=0)

    rm = jnp.max(jnp.max(iou3, axis=2), axis=1)
    rm3 = rm[:, None, None]
    cand3 = jnp.where(iou3 == rm3, pidx3, jnp.int32(PP))
    pa = jnp.min(jnp.min(cand3, axis=2), axis=1)
    pa3 = pa[:, None, None]
    msk3 = pidx3 == pa3
    forced = jnp.max(jnp.where(msk3, midx, -1), axis=0)
    has_forced = forced >= 0
    obj = jnp.where(has_forced, forced, obj)
    curmax = jnp.where(has_forced, 1.0, curmax)

    labp = jnp.zeros((SUB, LN), jnp.float32)
    gx1 = jnp.zeros((SUB, LN), jnp.float32)
    gy1 = jnp.zeros((SUB, LN), jnp.float32)
    gx2 = jnp.zeros((SUB, LN), jnp.float32)
    gy2 = jnp.zeros((SUB, LN), jnp.float32)
    for m in range(M):
        sel = obj == m
        labp = jnp.where(sel, lbl(m, 4), labp)
        gx1 = jnp.where(sel, lbl(m, 0), gx1)
        gy1 = jnp.where(sel, lbl(m, 1), gy1)
        gx2 = jnp.where(sel, lbl(m, 2), gx2)
        gy2 = jnp.where(sel, lbl(m, 3), gy2)
    labp = jnp.where(curmax < THRESHOLD, 0.0, labp)
    posf = (labp != 0.0).astype(jnp.float32)

    bcx = (gx1 + gx2) * 0.5
    bcy = (gy1 + gy2) * 0.5
    bw = gx2 - gx1
    bh = gy2 - gy1
    t0 = (bcx - PCX) * PIW10
    t1 = (bcy - PCY) * PIH10
    t2 = jnp.log(bw) * 5.0 - PLW5
    t3 = jnp.log(bh) * 5.0 - PLH5
    loc_v = (jnp.abs(ploc_ref[0, 0:8, :] - t0)
             + jnp.abs(ploc_ref[0, 8:16, :] - t1)
             + jnp.abs(ploc_ref[0, 16:24, :] - t2)
             + jnp.abs(ploc_ref[0, 24:32, :] - t3)) * posf

    @pl.when(b == 0)
    def _init():
        accloc_s[...] = loc_v

    @pl.when(b > 0)
    def _acc():
        accloc_s[...] = accloc_s[...] + loc_v

    def cls_plane(c):
        return pcls_ref[0, 8 * c:8 * c + 8, :]

    mx = cls_plane(0)
    for c in range(1, C):
        mx = jnp.maximum(mx, cls_plane(c))
    s = jnp.exp(cls_plane(0) - mx)
    for c in range(1, C):
        s = s + jnp.exp(cls_plane(c) - mx)
    logz = mx + jnp.log(s)
    pick = cls_plane(0)
    for c in range(1, C):
        pick = jnp.where(labp == jnp.float32(c), cls_plane(c), pick)

    conf_s[b] = logz - pick
    labp_s[b] = labp

    @pl.when(b == B - 1)
    def _fin():
        conf3 = conf_s[...]
        labp3 = labp_s[...]
        pos3 = labp3 != 0.0
        pos3f = pos3.astype(jnp.float32)
        valid3 = (pidx < P)[None, :, :]
        neg3 = jnp.where(pos3 | (~valid3), 0.0, conf3)
        bits = lax.bitcast_convert_type(neg3, jnp.int32)

        def rowsum(x):
            return jnp.sum(jnp.sum(x, axis=2), axis=1)

        npos_r = rowsum(pos3f)
        ki = (NEG_POS_RATIO * npos_r).astype(jnp.int32)

        def bit_step(i, t):
            cand = t | lax.shift_left(jnp.int32(1), 30 - i)
            ge = bits >= cand[:, None, None]
            cnt = rowsum(ge.astype(jnp.int32))
            return jnp.where(cnt >= ki, cand, t)

        tb = lax.fori_loop(0, 31, bit_step, jnp.zeros((B,), jnp.int32))
        tval = lax.bitcast_convert_type(tb, jnp.float32)
        gt = bits > tb[:, None, None]
        cgt = rowsum(gt.astype(jnp.int32))
        sgt = rowsum(jnp.where(gt, neg3, 0.0))
        hard_tot = jnp.sum(sgt + (ki - cgt).astype(jnp.float32) * tval)

        posc_tot = jnp.sum(jnp.sum(jnp.sum(conf3 * pos3f, axis=2), axis=1))
        npos_tot = jnp.sum(npos_r)
        loc_tot = jnp.sum(accloc_s[...])
        out_ref[0, 0] = ((hard_tot + posc_tot) / npos_tot
                         + ALPHA * loc_tot / (4.0 * npos_tot))


@jax.jit
def kernel(pred_loc, pred_cls, label):
    pl_t = jnp.transpose(pred_loc, (0, 2, 1))
    pl_r = jnp.pad(pl_t, ((0, 0), (0, 0), (0, PP - P)))
    pl_r = pl_r.reshape(B, 4 * SUB, LN)
    pc_t = jnp.transpose(pred_cls, (0, 2, 1))
    pc_r = jnp.pad(pc_t, ((0, 0), (0, 0), (0, PP - P)))
    pc_r = pc_r.reshape(B, C * SUB, LN)
    lbl_flat = label.reshape(B, 1, M * 5)

    out = pl.pallas_call(
        _body,
        grid=(B,),
        in_specs=[
            pl.BlockSpec((11 * SUB, LN), lambda b: (0, 0)),
            pl.BlockSpec((1, 1, M * 5), lambda b: (b, 0, 0), memory_space=pltpu.SMEM),
            pl.BlockSpec((1, 4 * SUB, LN), lambda b: (b, 0, 0)),
            pl.BlockSpec((1, C * SUB, LN), lambda b: (b, 0, 0)),
        ],
        out_specs=pl.BlockSpec((1, 1), lambda b: (0, 0), memory_space=pltpu.SMEM),
        out_shape=jax.ShapeDtypeStruct((1, 1), jnp.float32),
        scratch_shapes=[
            pltpu.VMEM((B, SUB, LN), jnp.float32),
            pltpu.VMEM((B, SUB, LN), jnp.float32),
            pltpu.VMEM((SUB, LN), jnp.float32),
        ],
        compiler_params=pltpu.CompilerParams(
            dimension_semantics=("arbitrary",)),
        interpret=_INTERPRET,
    )(_PR_PLANES, lbl_flat, pl_r, pc_r)
    return out[0, 0]
